# Initial kernel scaffold; baseline (speedup 1.0000x reference)
import functools

import jax
import jax.numpy as jnp
from jax import lax
from jax.experimental import pallas as pl
from jax.experimental.pallas import tpu as pltpu
from jax.experimental.pallas import tpu_sc as plsc

N_NODES = 10000
N_EDGES = 320000
D = 128
NC = 2
NS = 16
EPW = N_EDGES // (NC * NS)
C = 80
G = EPW // C
RPT = N_NODES // NS
N_PAD = 10240
CNT_PAD = 10240


def _sc_segment_sum(x, src, dst, z_acc, z_cnt):
    mesh = plsc.VectorSubcoreMesh(
        core_axis_name="c", subcore_axis_name="s", num_cores=NC, num_subcores=NS
    )

    @functools.partial(
        pl.kernel,
        out_type=[
            jax.ShapeDtypeStruct((NC, N_PAD, D), jnp.float32),
            jax.ShapeDtypeStruct((NC, CNT_PAD), jnp.float32),
        ],
        mesh=mesh,
        scratch_types=[
            pltpu.VMEM((G, C), jnp.int32),
            pltpu.VMEM((G, C), jnp.int32),
            pltpu.VMEM((C, D), jnp.float32),
            pltpu.VMEM((C,), jnp.float32),
            pltpu.VMEM_SHARED((N_NODES, D), jnp.float32),
            pltpu.VMEM_SHARED((CNT_PAD,), jnp.float32),
            pltpu.SemaphoreType.DMA,
        ],
    )
    def sc_kernel(x_hbm, src_hbm, dst_hbm, zacc_hbm, zcnt_hbm,
                  acc_out, cnt_out, src_v, dst_v, rows_v, ones_v,
                  acc_sh, cnt_sh, sem):
        c = lax.axis_index("c")
        s = lax.axis_index("s")

        pltpu.sync_copy(zacc_hbm, acc_sh.at[pl.ds(s * RPT, RPT)])

        @pl.when(s == 0)
        def _():
            pltpu.sync_copy(zcnt_hbm, cnt_sh)

        pltpu.sync_copy(src_hbm.at[c, s], src_v)
        pltpu.sync_copy(dst_hbm.at[c, s], dst_v)
        for j in range(C // 16):
            ones_v[pl.ds(j * 16, 16)] = jnp.full((16,), 1.0, jnp.float32)

        plsc.subcore_barrier()

        def chunk(g, carry):
            pltpu.async_copy(x_hbm.at[src_v.at[g]], rows_v, sem).wait()
            pltpu.sync_copy(rows_v, acc_sh.at[dst_v.at[g]], add=True)
            pltpu.sync_copy(ones_v, cnt_sh.at[dst_v.at[g]], add=True)
            return carry

        lax.fori_loop(0, G, chunk, 0)

        plsc.subcore_barrier()

        pltpu.sync_copy(acc_sh.at[pl.ds(s * RPT, RPT)],
                        acc_out.at[c, pl.ds(s * RPT, RPT)])

        @pl.when(s == 0)
        def _():
            pltpu.sync_copy(cnt_sh, cnt_out.at[c])

    return sc_kernel(x, src, dst, z_acc, z_cnt)


def _tc_combine(part, cnt3, x_pad, wt, b2):
    BN = 512
    grid = (N_PAD // BN,)

    def body(part_ref, cnt_ref, x_ref, wt_ref, b_ref, out_ref):
        csum = cnt_ref[0] + cnt_ref[1]
        neigh = (part_ref[0] + part_ref[1]) / jnp.maximum(csum, 1.0)
        comb = jnp.concatenate([neigh, x_ref[...]], axis=1)
        o = lax.dot_general(comb, wt_ref[...], (((1,), (0,)), ((), ())),
                            preferred_element_type=jnp.float32)
        o = o + b_ref[...]
        out_ref[...] = o * jax.nn.sigmoid(o)

    return pl.pallas_call(
        body,
        grid=grid,
        in_specs=[
            pl.BlockSpec((NC, BN, D), lambda i: (0, i, 0)),
            pl.BlockSpec((NC, BN, 1), lambda i: (0, i, 0)),
            pl.BlockSpec((BN, D), lambda i: (i, 0)),
            pl.BlockSpec((2 * D, D), lambda i: (0, 0)),
            pl.BlockSpec((1, D), lambda i: (0, 0)),
        ],
        out_specs=pl.BlockSpec((BN, D), lambda i: (i, 0)),
        out_shape=jax.ShapeDtypeStruct((N_PAD, D), jnp.float32),
    )(part, cnt3, x_pad, wt, b2)


def kernel(x, edge_index, W, b):
    ei = edge_index.astype(jnp.int32)
    src = ei[0].reshape(NC, NS, G, C)
    dst = ei[1].reshape(NC, NS, G, C)
    z_acc = jnp.zeros((RPT, D), jnp.float32)
    z_cnt = jnp.zeros((CNT_PAD,), jnp.float32)

    part, cnt = _sc_segment_sum(x, src, dst, z_acc, z_cnt)

    x_pad = jnp.pad(x, ((0, N_PAD - N_NODES), (0, 0)))
    cnt3 = cnt.reshape(NC, CNT_PAD, 1)
    wt = W.T
    b2 = b.reshape(1, D)
    out = _tc_combine(part, cnt3, x_pad, wt, b2)
    return out[:N_NODES]


# speedup vs baseline: 7.8875x; 7.8875x over previous
import functools

import jax
import jax.numpy as jnp
from jax import lax
from jax.experimental import pallas as pl
from jax.experimental.pallas import tpu as pltpu
from jax.experimental.pallas import tpu_sc as plsc

N_NODES = 10000
N_EDGES = 320000
D = 128
NC = 2
NS = 16
EPW = N_EDGES // (NC * NS)
C = 80
G = EPW // C
N_PAD = 10240
RPT = N_PAD // NS
CNT_PAD = 10240


def _sc_segment_sum(x, src, dst, z_acc, z_cnt):
    mesh = plsc.VectorSubcoreMesh(
        core_axis_name="c", subcore_axis_name="s", num_cores=NC, num_subcores=NS
    )

    @functools.partial(
        pl.kernel,
        out_type=[
            jax.ShapeDtypeStruct((NC, N_PAD, D), jnp.float32),
            jax.ShapeDtypeStruct((NC, CNT_PAD), jnp.float32),
        ],
        mesh=mesh,
        scratch_types=[
            pltpu.VMEM((G, C), jnp.int32),
            pltpu.VMEM((G, C), jnp.int32),
            pltpu.VMEM((C, D), jnp.float32),
            pltpu.VMEM((C,), jnp.float32),
            pltpu.VMEM_SHARED((N_PAD, D), jnp.float32),
            pltpu.VMEM_SHARED((CNT_PAD,), jnp.float32),
            pltpu.SemaphoreType.DMA,
        ],
    )
    def sc_kernel(x_hbm, src_hbm, dst_hbm, zacc_hbm, zcnt_hbm,
                  acc_out, cnt_out, src_v, dst_v, rows_v, ones_v,
                  acc_sh, cnt_sh, sem):
        c = lax.axis_index("c")
        s = lax.axis_index("s")

        pltpu.sync_copy(zacc_hbm, acc_sh.at[pl.ds(s * RPT, RPT)])

        @pl.when(s == 0)
        def _():
            pltpu.sync_copy(zcnt_hbm, cnt_sh)

        pltpu.sync_copy(src_hbm.at[c, s], src_v)
        pltpu.sync_copy(dst_hbm.at[c, s], dst_v)
        for j in range(C // 16):
            ones_v[pl.ds(j * 16, 16)] = jnp.full((16,), 1.0, jnp.float32)

        plsc.subcore_barrier()

        def chunk(g, carry):
            pltpu.async_copy(x_hbm.at[src_v.at[g]], rows_v, sem).wait()
            pltpu.sync_copy(rows_v, acc_sh.at[dst_v.at[g]], add=True)
            pltpu.sync_copy(ones_v, cnt_sh.at[dst_v.at[g]], add=True)
            return carry

        lax.fori_loop(0, G, chunk, 0)

        plsc.subcore_barrier()

        pltpu.sync_copy(acc_sh.at[pl.ds(s * RPT, RPT)],
                        acc_out.at[c, pl.ds(s * RPT, RPT)])

        @pl.when(s == 0)
        def _():
            pltpu.sync_copy(cnt_sh, cnt_out.at[c])

    return sc_kernel(x, src, dst, z_acc, z_cnt)


def _tc_combine(part, cnt3, x_pad, wt, b2):
    BN = 512
    grid = (N_PAD // BN,)

    def body(part_ref, cnt_ref, x_ref, wt_ref, b_ref, out_ref):
        csum = cnt_ref[0] + cnt_ref[1]
        neigh = (part_ref[0] + part_ref[1]) / jnp.maximum(csum, 1.0)
        comb = jnp.concatenate([neigh, x_ref[...]], axis=1)
        o = lax.dot_general(comb, wt_ref[...], (((1,), (0,)), ((), ())),
                            preferred_element_type=jnp.float32)
        o = o + b_ref[...]
        out_ref[...] = o * jax.nn.sigmoid(o)

    return pl.pallas_call(
        body,
        grid=grid,
        in_specs=[
            pl.BlockSpec((NC, BN, D), lambda i: (0, i, 0)),
            pl.BlockSpec((NC, BN, 1), lambda i: (0, i, 0)),
            pl.BlockSpec((BN, D), lambda i: (i, 0)),
            pl.BlockSpec((2 * D, D), lambda i: (0, 0)),
            pl.BlockSpec((1, D), lambda i: (0, 0)),
        ],
        out_specs=pl.BlockSpec((BN, D), lambda i: (i, 0)),
        out_shape=jax.ShapeDtypeStruct((N_PAD, D), jnp.float32),
    )(part, cnt3, x_pad, wt, b2)


def kernel(x, edge_index, W, b):
    ei = edge_index.astype(jnp.int32)
    src = ei[0].reshape(NC, NS, G, C)
    dst = ei[1].reshape(NC, NS, G, C)
    z_acc = jnp.zeros((RPT, D), jnp.float32)
    z_cnt = jnp.zeros((CNT_PAD,), jnp.float32)

    part, cnt = _sc_segment_sum(x, src, dst, z_acc, z_cnt)

    x_pad = jnp.pad(x, ((0, N_PAD - N_NODES), (0, 0)))
    cnt3 = cnt.reshape(NC, CNT_PAD, 1)
    wt = W.T
    b2 = b.reshape(1, D)
    out = _tc_combine(part, cnt3, x_pad, wt, b2)
    return out[:N_NODES]


# speedup vs baseline: 11.6496x; 1.4770x over previous
import functools

import jax
import jax.numpy as jnp
from jax import lax
from jax.experimental import pallas as pl
from jax.experimental.pallas import tpu as pltpu
from jax.experimental.pallas import tpu_sc as plsc

N_NODES = 10000
N_EDGES = 320000
D = 128
NC = 2
NS = 16
C = 128
G = 80
E_PAD = NC * NS * G * C
N_PAD = 10240
RPT = N_PAD // NS
CNT_PAD = 10240


def _sc_segment_sum(x, src, dst, z_acc, z_cnt):
    mesh = plsc.VectorSubcoreMesh(
        core_axis_name="c", subcore_axis_name="s", num_cores=NC, num_subcores=NS
    )

    @functools.partial(
        pl.kernel,
        out_type=[
            jax.ShapeDtypeStruct((NC, N_PAD, D), jnp.float32),
            jax.ShapeDtypeStruct((NC, CNT_PAD), jnp.float32),
        ],
        mesh=mesh,
        scratch_types=[
            pltpu.VMEM((G, C), jnp.int32),
            pltpu.VMEM((2, C), jnp.int32),
            pltpu.VMEM((C, D), jnp.float32),
            pltpu.VMEM((C, D), jnp.float32),
            pltpu.VMEM((C,), jnp.float32),
            pltpu.VMEM_SHARED((N_PAD, D), jnp.float32),
            pltpu.VMEM_SHARED((CNT_PAD,), jnp.float32),
            pltpu.SemaphoreType.DMA,
            pltpu.SemaphoreType.DMA,
            pltpu.SemaphoreType.DMA,
            pltpu.SemaphoreType.DMA,
        ],
    )
    def sc_kernel(x_hbm, src_hbm, dst_hbm, zacc_hbm, zcnt_hbm,
                  acc_out, cnt_out, dst_v, sidx_v, rows0_v, rows1_v, ones_v,
                  acc_sh, cnt_sh, semi0, semi1, semg0, semg1):
        c = lax.axis_index("c")
        s = lax.axis_index("s")

        pltpu.sync_copy(zacc_hbm, acc_sh.at[pl.ds(s * RPT, RPT)])

        @pl.when(s == 0)
        def _():
            pltpu.sync_copy(zcnt_hbm, cnt_sh)

        pltpu.sync_copy(dst_hbm.at[c, s], dst_v)
        for j in range(C // 16):
            ones_v[pl.ds(j * 16, 16)] = jnp.full((16,), 1.0, jnp.float32)

        plsc.subcore_barrier()

        def istart(g, slot, sem):
            pltpu.async_copy(src_hbm.at[c, s, g], slot, sem)

        def idrain(slot, sem):
            pltpu.make_async_copy(src_hbm.at[c, s, 0], slot, sem).wait()

        def gstart(slot, buf, sem):
            pltpu.async_copy(x_hbm.at[slot], buf, sem)

        def gdrain(buf, sem):
            pltpu.make_async_copy(x_hbm.at[pl.ds(0, C)], buf, sem).wait()

        def scat(buf, g):
            pltpu.sync_copy(buf, acc_sh.at[dst_v.at[g]], add=True)
            pltpu.sync_copy(ones_v, cnt_sh.at[dst_v.at[g]], add=True)

        istart(0, sidx_v.at[0], semi0)
        idrain(sidx_v.at[0], semi0)
        gstart(sidx_v.at[0], rows0_v, semg0)
        istart(1, sidx_v.at[1], semi1)

        def chunk2(k, carry):
            g = 2 * k
            gdrain(rows0_v, semg0)

            @pl.when(g + 2 < G)
            def _():
                istart(g + 2, sidx_v.at[0], semi0)

            idrain(sidx_v.at[1], semi1)
            gstart(sidx_v.at[1], rows1_v, semg1)
            scat(rows0_v, g)
            gdrain(rows1_v, semg1)

            @pl.when(g + 3 < G)
            def _():
                istart(g + 3, sidx_v.at[1], semi1)

            @pl.when(g + 2 < G)
            def _():
                idrain(sidx_v.at[0], semi0)
                gstart(sidx_v.at[0], rows0_v, semg0)

            scat(rows1_v, g + 1)
            return carry

        lax.fori_loop(0, G // 2, chunk2, 0)

        plsc.subcore_barrier()

        pltpu.sync_copy(acc_sh.at[pl.ds(s * RPT, RPT)],
                        acc_out.at[c, pl.ds(s * RPT, RPT)])

        @pl.when(s == 0)
        def _():
            pltpu.sync_copy(cnt_sh, cnt_out.at[c])

    return sc_kernel(x, src, dst, z_acc, z_cnt)


def _tc_combine(part, cnt3, x_pad, wt, b2):
    BN = 512
    grid = (N_PAD // BN,)

    def body(part_ref, cnt_ref, x_ref, wt_ref, b_ref, out_ref):
        csum = cnt_ref[0] + cnt_ref[1]
        neigh = (part_ref[0] + part_ref[1]) / jnp.maximum(csum, 1.0)
        comb = jnp.concatenate([neigh, x_ref[...]], axis=1)
        o = lax.dot_general(comb, wt_ref[...], (((1,), (0,)), ((), ())),
                            preferred_element_type=jnp.float32)
        o = o + b_ref[...]
        out_ref[...] = o * jax.nn.sigmoid(o)

    return pl.pallas_call(
        body,
        grid=grid,
        in_specs=[
            pl.BlockSpec((NC, BN, D), lambda i: (0, i, 0)),
            pl.BlockSpec((NC, BN, 1), lambda i: (0, i, 0)),
            pl.BlockSpec((BN, D), lambda i: (i, 0)),
            pl.BlockSpec((2 * D, D), lambda i: (0, 0)),
            pl.BlockSpec((1, D), lambda i: (0, 0)),
        ],
        out_specs=pl.BlockSpec((BN, D), lambda i: (i, 0)),
        out_shape=jax.ShapeDtypeStruct((N_PAD, D), jnp.float32),
    )(part, cnt3, x_pad, wt, b2)


def kernel(x, edge_index, W, b):
    ei = edge_index.astype(jnp.int32)
    npad = E_PAD - N_EDGES
    pad_src = (jnp.arange(npad, dtype=jnp.int32) * 131) % N_NODES
    pad_dst = N_NODES + (jnp.arange(npad, dtype=jnp.int32) % (N_PAD - N_NODES))
    src = jnp.concatenate([ei[0], pad_src]).reshape(NC, NS, G, C)
    dst = jnp.concatenate([ei[1], pad_dst]).reshape(NC, NS, G, C)
    z_acc = jnp.zeros((RPT, D), jnp.float32)
    z_cnt = jnp.zeros((CNT_PAD,), jnp.float32)

    part, cnt = _sc_segment_sum(x, src, dst, z_acc, z_cnt)

    x_pad = jnp.pad(x, ((0, N_PAD - N_NODES), (0, 0)))
    cnt3 = cnt.reshape(NC, CNT_PAD, 1)
    wt = W.T
    b2 = b.reshape(1, D)
    out = _tc_combine(part, cnt3, x_pad, wt, b2)
    return out[:N_NODES]


# speedup vs baseline: 11.7645x; 1.0099x over previous
import functools

import jax
import jax.numpy as jnp
import numpy as np
from jax import lax
from jax.experimental import pallas as pl
from jax.experimental.pallas import tpu as pltpu
from jax.experimental.pallas import tpu_sc as plsc

N_NODES = 10000
N_EDGES = 320000
D = 128
NC = 2
NS = 16
C = 128
G = 80
E_PAD = NC * NS * G * C
N_PAD = 10240
RPT = N_PAD // NS
CNT_PAD = 10240


def _sc_segment_sum(x, src, dst, z_acc, z_cnt):
    mesh = plsc.VectorSubcoreMesh(
        core_axis_name="c", subcore_axis_name="s", num_cores=NC, num_subcores=NS
    )

    @functools.partial(
        pl.kernel,
        out_type=[
            jax.ShapeDtypeStruct((NC, N_PAD, D), jnp.float32),
            jax.ShapeDtypeStruct((NC, CNT_PAD), jnp.float32),
        ],
        mesh=mesh,
        scratch_types=[
            pltpu.VMEM((G, C), jnp.int32),
            pltpu.VMEM((2, C), jnp.int32),
            pltpu.VMEM((C, D), jnp.float32),
            pltpu.VMEM((C, D), jnp.float32),
            pltpu.VMEM((C,), jnp.float32),
            pltpu.VMEM_SHARED((N_PAD, D), jnp.float32),
            pltpu.VMEM_SHARED((CNT_PAD,), jnp.float32),
            pltpu.SemaphoreType.DMA,
            pltpu.SemaphoreType.DMA,
            pltpu.SemaphoreType.DMA,
            pltpu.SemaphoreType.DMA,
            pltpu.SemaphoreType.DMA,
        ],
    )
    def sc_kernel(x_hbm, src_hbm, dst_hbm, zacc_hbm, zcnt_hbm,
                  acc_out, cnt_out, dst_v, sidx_v, rows0_v, rows1_v, ones_v,
                  acc_sh, cnt_sh, semi0, semi1, semg0, semg1, semc):
        c = lax.axis_index("c")
        s = lax.axis_index("s")

        pltpu.sync_copy(zacc_hbm, acc_sh.at[pl.ds(s * RPT, RPT)])

        @pl.when(s == 0)
        def _():
            pltpu.sync_copy(zcnt_hbm, cnt_sh)

        pltpu.sync_copy(dst_hbm.at[c, s], dst_v)
        for j in range(C // 16):
            ones_v[pl.ds(j * 16, 16)] = jnp.full((16,), 1.0, jnp.float32)

        plsc.subcore_barrier()

        def istart(g, slot, sem):
            pltpu.async_copy(src_hbm.at[c, s, g], slot, sem)

        def idrain(slot, sem):
            pltpu.make_async_copy(src_hbm.at[c, s, 0], slot, sem).wait()

        def gstart(slot, buf, sem):
            pltpu.async_copy(x_hbm.at[slot], buf, sem)

        def gdrain(buf, sem):
            pltpu.make_async_copy(x_hbm.at[pl.ds(0, C)], buf, sem).wait()

        def scat(buf, g):
            pltpu.sync_copy(buf, acc_sh.at[dst_v.at[g]], add=True)
            pltpu.async_copy(ones_v, cnt_sh.at[dst_v.at[g]], semc, add=True)

        istart(0, sidx_v.at[0], semi0)
        idrain(sidx_v.at[0], semi0)
        gstart(sidx_v.at[0], rows0_v, semg0)
        istart(1, sidx_v.at[1], semi1)

        def chunk2(k, carry):
            g = 2 * k
            gdrain(rows0_v, semg0)

            @pl.when(g + 2 < G)
            def _():
                istart(g + 2, sidx_v.at[0], semi0)

            idrain(sidx_v.at[1], semi1)
            gstart(sidx_v.at[1], rows1_v, semg1)
            scat(rows0_v, g)
            gdrain(rows1_v, semg1)

            @pl.when(g + 3 < G)
            def _():
                istart(g + 3, sidx_v.at[1], semi1)

            @pl.when(g + 2 < G)
            def _():
                idrain(sidx_v.at[0], semi0)
                gstart(sidx_v.at[0], rows0_v, semg0)

            scat(rows1_v, g + 1)
            return carry

        lax.fori_loop(0, G // 2, chunk2, 0)

        def cdrain(g, carry):
            pltpu.make_async_copy(zcnt_hbm.at[pl.ds(0, C)], ones_v, semc).wait()
            return carry

        lax.fori_loop(0, G, cdrain, 0)

        plsc.subcore_barrier()

        pltpu.sync_copy(acc_sh.at[pl.ds(s * RPT, RPT)],
                        acc_out.at[c, pl.ds(s * RPT, RPT)])

        @pl.when(s == 0)
        def _():
            pltpu.sync_copy(cnt_sh, cnt_out.at[c])

    return sc_kernel(x, src, dst, z_acc, z_cnt)


def _tc_combine(part, cnt3, x, wt, b2):
    BN = 400
    grid = (N_NODES // BN,)

    def body(part_ref, cnt_ref, x_ref, wt_ref, b_ref, out_ref):
        csum = cnt_ref[0] + cnt_ref[1]
        neigh = (part_ref[0] + part_ref[1]) / jnp.maximum(csum, 1.0)
        comb = jnp.concatenate([neigh, x_ref[...]], axis=1)
        o = lax.dot_general(comb, wt_ref[...], (((1,), (0,)), ((), ())),
                            preferred_element_type=jnp.float32)
        o = o + b_ref[...]
        out_ref[...] = o * jax.nn.sigmoid(o)

    return pl.pallas_call(
        body,
        grid=grid,
        in_specs=[
            pl.BlockSpec((NC, BN, D), lambda i: (0, i, 0)),
            pl.BlockSpec((NC, BN, 1), lambda i: (0, i, 0)),
            pl.BlockSpec((BN, D), lambda i: (i, 0)),
            pl.BlockSpec((2 * D, D), lambda i: (0, 0)),
            pl.BlockSpec((1, D), lambda i: (0, 0)),
        ],
        out_specs=pl.BlockSpec((BN, D), lambda i: (i, 0)),
        out_shape=jax.ShapeDtypeStruct((N_NODES, D), jnp.float32),
    )(part, cnt3, x, wt, b2)


_NPADE = E_PAD - N_EDGES
_PAD_SRC = np.asarray((np.arange(_NPADE) * 131) % N_NODES, dtype=np.int32)
_PAD_DST = np.asarray(N_NODES + (np.arange(_NPADE) % (N_PAD - N_NODES)),
                      dtype=np.int32)


def kernel(x, edge_index, W, b):
    ei = edge_index.astype(jnp.int32)
    src = jnp.concatenate([ei[0], _PAD_SRC]).reshape(NC, NS, G, C)
    dst = jnp.concatenate([ei[1], _PAD_DST]).reshape(NC, NS, G, C)
    z_acc = jnp.zeros((RPT, D), jnp.float32)
    z_cnt = jnp.zeros((CNT_PAD,), jnp.float32)

    part, cnt = _sc_segment_sum(x, src, dst, z_acc, z_cnt)

    cnt3 = cnt.reshape(NC, CNT_PAD, 1)
    wt = W.T
    b2 = b.reshape(1, D)
    return _tc_combine(part, cnt3, x, wt, b2)
